# Initial kernel scaffold; baseline (speedup 1.0000x reference)
import jax
import jax.numpy as jnp
from jax.experimental import pallas as pl


def kernel(x, edge_index, edge_attr, W1l, W1r, a1, We1, b1, W2l, W2r, a2, We2, b2, W3l, W3r, a3, We3, b3, g1, be1, g2, be2, g3, be3, D1, d1b, ga, ba, D2, d2b, gb, bb, D3, d3b):
    raise NotImplementedError("write your pallas kernel here")


# speedup vs baseline: 10.8688x; 10.8688x over previous
import functools

import jax
import jax.numpy as jnp
from jax import lax
from jax.experimental import pallas as pl
from jax.experimental.pallas import tpu as pltpu

_N = 1024
_E = 16384
_ET = _E + _N
_BE = 512
_NB = _ET // _BE
_BI = 8
_NBI = _N // _BI
_EPS = 1e-5


def _lrelu(v, slope):
    return jnp.where(v >= 0, v, slope * v)


def _gat_layer(x, s_cols, d_cols, d_rows, ea_cols, ear, Wl, Wr, Att, Exp, We,
               bias, g, be, do_elu):
    F = x.shape[1]
    C = Wl.shape[1]
    H = Att.shape[1]

    def body(s_ref, dc_ref, dr_ref, ea_ref, ear_ref, x_ref, wl_ref, wr_ref,
             att_ref, exp_ref, we_ref, b_ref, g_ref, be_ref, out_ref,
             xl_s, xr_s, num_s, den_s, mean_s):
        k = pl.program_id(0)

        @pl.when(k == 0)
        def _():
            xv = x_ref[...]
            xl_s[...] = jnp.dot(xv, wl_ref[...],
                                preferred_element_type=jnp.float32)
            xr_s[...] = jnp.dot(xv, wr_ref[...],
                                preferred_element_type=jnp.float32)
            num_s[...] = jnp.zeros_like(num_s)
            den_s[...] = jnp.zeros_like(den_s)
            mean_s[0, 0] = jnp.sum(ear_ref[...]) / _E

        s = s_ref[0]
        dcol = dc_ref[0]
        drow = dr_ref[0]
        eid = k * _BE + lax.broadcasted_iota(jnp.int32, (_BE, 1), 0)
        ea = jnp.where(eid < _E, ea_ref[0], mean_s[0, 0])

        iota_en = lax.broadcasted_iota(jnp.int32, (_BE, _N), 1)
        iota_ne = lax.broadcasted_iota(jnp.int32, (_N, _BE), 0)
        Gs = (s == iota_en).astype(jnp.float32)
        Gd = (dcol == iota_en).astype(jnp.float32)
        Gdt = (drow == iota_ne).astype(jnp.float32)

        xle = jnp.dot(Gs, xl_s[...], preferred_element_type=jnp.float32)
        xre = jnp.dot(Gd, xr_s[...], preferred_element_type=jnp.float32)
        m = _lrelu(xle + xre + ea * we_ref[...], 0.2)
        logits = jnp.dot(m, att_ref[...], preferred_element_type=jnp.float32)
        exl = jnp.exp(logits)
        exl_c = jnp.dot(exl, exp_ref[...], preferred_element_type=jnp.float32)
        num_s[...] += jnp.dot(Gdt, xle * exl_c,
                              preferred_element_type=jnp.float32)
        den_s[...] += jnp.dot(Gdt, exl, preferred_element_type=jnp.float32)

        @pl.when(k == _NB - 1)
        def _():
            den_c = jnp.dot(den_s[...], exp_ref[...],
                            preferred_element_type=jnp.float32)
            o = num_s[...] / den_c + b_ref[...]
            mu = jnp.mean(o, axis=-1, keepdims=True)
            var = jnp.mean((o - mu) * (o - mu), axis=-1, keepdims=True)
            o = (o - mu) * lax.rsqrt(var + _EPS) * g_ref[...] + be_ref[...]
            if do_elu:
                o = jnp.where(o > 0, o, jnp.exp(o) - 1.0)
            out_ref[...] = o

    full = lambda shape: pl.BlockSpec(shape, lambda k: tuple(0 for _ in shape))
    return pl.pallas_call(
        body,
        grid=(_NB,),
        in_specs=[
            pl.BlockSpec((1, _BE, 1), lambda k: (k, 0, 0)),
            pl.BlockSpec((1, _BE, 1), lambda k: (k, 0, 0)),
            pl.BlockSpec((1, 1, _BE), lambda k: (k, 0, 0)),
            pl.BlockSpec((1, _BE, 1), lambda k: (k, 0, 0)),
            full((1, _E)),
            full((_N, F)),
            full((F, C)),
            full((F, C)),
            full((C, H)),
            full((H, C)),
            full((1, C)),
            full((1, C)),
            full((1, C)),
            full((1, C)),
        ],
        out_specs=full((_N, C)),
        out_shape=jax.ShapeDtypeStruct((_N, C), jnp.float32),
        scratch_shapes=[
            pltpu.VMEM((_N, C), jnp.float32),
            pltpu.VMEM((_N, C), jnp.float32),
            pltpu.VMEM((_N, C), jnp.float32),
            pltpu.VMEM((_N, H), jnp.float32),
            pltpu.SMEM((1, 1), jnp.float32),
        ],
        compiler_params=pltpu.CompilerParams(
            dimension_semantics=("arbitrary",)),
    )(s_cols, d_cols, d_rows, ea_cols, ear, x, Wl, Wr, Att, Exp, We,
      bias, g, be)


def _decode(embT, D1aT, D1bT, d1b_c, ga_c, ba_c, D2T, d2b_c, gb_c, bb_c,
            D3_c, d3b_c):
    def body(embT_ref, d1a_ref, d1b_ref, d1bias_ref, ga_ref, ba_ref, d2_ref,
             d2b_ref, gb_ref, bb_ref, d3_ref, d3b_ref, out_ref, A_s, B_s):
        k = pl.program_id(0)

        @pl.when(k == 0)
        def _():
            et = embT_ref[...]
            A_s[...] = jnp.dot(d1a_ref[...], et,
                               preferred_element_type=jnp.float32) \
                + d1bias_ref[...]
            B_s[...] = jnp.dot(d1b_ref[...], et,
                               preferred_element_type=jnp.float32)

        B = B_s[...]
        Afull = A_s[...]
        ga_v = ga_ref[...]
        ba_v = ba_ref[...]
        gb_v = gb_ref[...]
        bb_v = bb_ref[...]
        d2b_v = d2b_ref[...]
        d3_v = d3_ref[...]
        iota_cols = lax.broadcasted_iota(jnp.int32, (64, _N), 1)
        for r in range(_BI):
            sel = iota_cols == (k * _BI + r)
            a = jnp.sum(jnp.where(sel, Afull, 0.0), axis=1, keepdims=True)
            h = B + a
            mu = jnp.mean(h, axis=0, keepdims=True)
            var = jnp.mean((h - mu) * (h - mu), axis=0, keepdims=True)
            h = (h - mu) * lax.rsqrt(var + _EPS) * ga_v + ba_v
            h = _lrelu(h, 0.1)
            h2 = jnp.dot(d2_ref[...], h,
                         preferred_element_type=jnp.float32) + d2b_v
            mu2 = jnp.mean(h2, axis=0, keepdims=True)
            var2 = jnp.mean((h2 - mu2) * (h2 - mu2), axis=0, keepdims=True)
            h2 = (h2 - mu2) * lax.rsqrt(var2 + _EPS) * gb_v + bb_v
            h2 = _lrelu(h2, 0.1)
            logit = jnp.sum(h2 * d3_v, axis=0, keepdims=True) + d3b_ref[...]
            out_ref[r:r + 1, :] = 1.0 / (1.0 + jnp.exp(-logit))

    full = lambda shape: pl.BlockSpec(shape, lambda k: tuple(0 for _ in shape))
    return pl.pallas_call(
        body,
        grid=(_NBI,),
        in_specs=[
            full((32, _N)),
            full((64, 32)),
            full((64, 32)),
            full((64, 1)),
            full((64, 1)),
            full((64, 1)),
            full((32, 64)),
            full((32, 1)),
            full((32, 1)),
            full((32, 1)),
            full((32, 1)),
            full((1, 1)),
        ],
        out_specs=pl.BlockSpec((_BI, _N), lambda k: (k, 0)),
        out_shape=jax.ShapeDtypeStruct((_N, _N), jnp.float32),
        scratch_shapes=[
            pltpu.VMEM((64, _N), jnp.float32),
            pltpu.VMEM((64, _N), jnp.float32),
        ],
        compiler_params=pltpu.CompilerParams(
            dimension_semantics=("arbitrary",)),
    )(embT, D1aT, D1bT, d1b_c, ga_c, ba_c, D2T, d2b_c, gb_c, bb_c,
      D3_c, d3b_c)


def _att_mats(a, heads, ch):
    C = heads * ch
    mask = (jnp.arange(C)[:, None] // ch) == jnp.arange(heads)[None, :]
    Att = jnp.where(mask, a.reshape(C, 1), 0.0).astype(jnp.float32)
    Exp = mask.T.astype(jnp.float32)
    return Att, Exp


@jax.jit
def kernel(x, edge_index, edge_attr, W1l, W1r, a1, We1, b1, W2l, W2r, a2, We2,
           b2, W3l, W3r, a3, We3, b3, g1, be1, g2, be2, g3, be3, D1, d1b, ga,
           ba, D2, d2b, gb, bb, D3, d3b):
    src, dst = edge_index[0], edge_index[1]
    loop = jnp.arange(_N, dtype=src.dtype)
    s2 = jnp.concatenate([src, loop])
    d2 = jnp.concatenate([dst, loop])
    ea2 = jnp.concatenate([edge_attr[:, 0], jnp.zeros(_N, jnp.float32)])

    s_cols = s2.reshape(_NB, _BE, 1)
    d_cols = d2.reshape(_NB, _BE, 1)
    d_rows = d2.reshape(_NB, 1, _BE)
    ea_cols = ea2.reshape(_NB, _BE, 1)
    ear = edge_attr.reshape(1, _E)

    row = lambda v: v.reshape(1, -1)
    Att1, Exp1 = _att_mats(a1, 4, 16)
    Att2, Exp2 = _att_mats(a2, 4, 16)
    Att3, Exp3 = _att_mats(a3, 1, 32)

    h = _gat_layer(x, s_cols, d_cols, d_rows, ea_cols, ear, W1l, W1r,
                   Att1, Exp1, We1, row(b1), row(g1), row(be1), True)
    h = _gat_layer(h, s_cols, d_cols, d_rows, ea_cols, ear, W2l, W2r,
                   Att2, Exp2, We2, row(b2), row(g2), row(be2), True)
    emb = _gat_layer(h, s_cols, d_cols, d_rows, ea_cols, ear, W3l, W3r,
                     Att3, Exp3, We3, row(b3), row(g3), row(be3), False)

    col = lambda v: v.reshape(-1, 1)
    weights = _decode(emb.T, D1[:32].T, D1[32:].T, col(d1b), col(ga), col(ba),
                      D2.T, col(d2b), col(gb), col(bb), D3.reshape(32, 1),
                      d3b.reshape(1, 1))
    return (weights, emb)
